# baseline (speedup 1.0000x reference)
import functools

import jax
import jax.numpy as jnp
from jax import lax
from jax.experimental import pallas as pl
from jax.experimental.pallas import tpu as pltpu
from jax.experimental.pallas import tpu_sc as plsc

NC, NS, L = 2, 16, 16
NW = NC * NS
N = 2048 * 4096
PER_W = N // NW
CHUNK = 16384
NCHUNK = PER_W // CHUNK

_mesh = plsc.VectorSubcoreMesh(
    core_axis_name="c", subcore_axis_name="s", num_cores=NC, num_subcores=NS
)


@functools.partial(
    pl.kernel,
    mesh=_mesh,
    out_type=jax.ShapeDtypeStruct((N,), jnp.float32),
    scratch_types=[
        pltpu.VMEM((CHUNK,), jnp.float32),
        pltpu.VMEM((CHUNK,), jnp.int32),
        pltpu.VMEM((CHUNK,), jnp.float32),
    ],
)
def _cordiv_sc(div_hbm, dsr_hbm, out_hbm, a_v, b_v, o_v):
    wid = lax.axis_index("s") * NC + lax.axis_index("c")
    base = wid * PER_W

    def chunk_body(g, carry):
        off = pl.multiple_of(base + g * CHUNK, CHUNK)
        pltpu.sync_copy(div_hbm.at[pl.ds(off, CHUNK)], a_v)
        pltpu.sync_copy(dsr_hbm.at[pl.ds(off, CHUNK)], b_v)

        def vec_body(i, c):
            s = i * L
            d = a_v[pl.ds(s, L)]
            q = b_v[pl.ds(s, L)]
            o_v[pl.ds(s, L)] = jnp.where(q == 1, d, jnp.zeros((L,), jnp.float32))
            return c

        lax.fori_loop(0, CHUNK // L, vec_body, 0)
        pltpu.sync_copy(o_v, out_hbm.at[pl.ds(off, CHUNK)])
        return carry

    lax.fori_loop(0, NCHUNK, chunk_body, 0)


def kernel(dividend, divisor):
    out = _cordiv_sc(dividend.reshape(-1), divisor.reshape(-1))
    return out.reshape(dividend.shape)


# speedup vs baseline: 1.3525x; 1.3525x over previous
import functools

import jax
import jax.numpy as jnp
from jax import lax
from jax.experimental import pallas as pl
from jax.experimental.pallas import tpu as pltpu
from jax.experimental.pallas import tpu_sc as plsc

NC, NS, L = 2, 16, 16
NW = NC * NS
N = 2048 * 4096
PER_W = N // NW
CHUNK = 16384
NCHUNK = PER_W // CHUNK

_mesh = plsc.VectorSubcoreMesh(
    core_axis_name="c", subcore_axis_name="s", num_cores=NC, num_subcores=NS
)


@functools.partial(
    pl.kernel,
    mesh=_mesh,
    out_type=jax.ShapeDtypeStruct((N,), jnp.float32),
    scratch_types=[
        pltpu.VMEM((2, CHUNK), jnp.float32),
        pltpu.VMEM((2, CHUNK), jnp.int32),
        pltpu.VMEM((2, CHUNK), jnp.float32),
        pltpu.SemaphoreType.DMA,
        pltpu.SemaphoreType.DMA,
        pltpu.SemaphoreType.DMA,
        pltpu.SemaphoreType.DMA,
        pltpu.SemaphoreType.DMA,
        pltpu.SemaphoreType.DMA,
    ],
)
def _cordiv_sc(div_hbm, dsr_hbm, out_hbm, a_v, b_v, o_v, sa0, sa1, sb0, sb1, so0, so1):
    wid = lax.axis_index("s") * NC + lax.axis_index("c")
    base = wid * PER_W
    sa = (sa0, sa1)
    sb = (sb0, sb1)
    so = (so0, so1)
    zeros = jnp.zeros((L,), jnp.float32)

    def in_copies(g, s):
        off = base + g * CHUNK
        ca = pltpu.async_copy(div_hbm.at[pl.ds(off, CHUNK)], a_v.at[s], sa[s])
        cb = pltpu.async_copy(dsr_hbm.at[pl.ds(off, CHUNK)], b_v.at[s], sb[s])
        return ca, cb

    def out_copy(g, s):
        off = base + g * CHUNK
        return pltpu.async_copy(o_v.at[s], out_hbm.at[pl.ds(off, CHUNK)], so[s])

    pend_in = [None, None]
    pend_out = [None, None]
    pend_in[0] = in_copies(0, 0)
    for g in range(NCHUNK):
        s = g % 2
        if g + 1 < NCHUNK:
            pend_in[1 - s] = in_copies(g + 1, 1 - s)
        ca, cb = pend_in[s]
        ca.wait()
        cb.wait()
        if pend_out[s] is not None:
            pend_out[s].wait()
        a_s = a_v.at[s]
        b_s = b_v.at[s]
        o_s = o_v.at[s]

        @plsc.parallel_loop(0, CHUNK, step=L, unroll=8)
        def _(i):
            d = a_s[pl.ds(i, L)]
            q = b_s[pl.ds(i, L)]
            o_s[pl.ds(i, L)] = jnp.where(q == 1, d, zeros)

        pend_out[s] = out_copy(g, s)

    pend_out[0].wait()
    pend_out[1].wait()


def kernel(dividend, divisor):
    out = _cordiv_sc(dividend.reshape(-1), divisor.reshape(-1))
    return out.reshape(dividend.shape)


# speedup vs baseline: 3.6087x; 2.6682x over previous
import functools

import jax
import jax.numpy as jnp
from jax import lax
from jax.experimental import pallas as pl
from jax.experimental.pallas import tpu as pltpu
from jax.experimental.pallas import tpu_sc as plsc

NC, NS, L = 2, 16, 16
NW = NC * NS
ROWS, COLS = 2048, 4096
ROWS_W = ROWS // NW
BR, BC = 8, 2048
NBLK = (ROWS_W // BR) * (COLS // BC)

_mesh = plsc.VectorSubcoreMesh(
    core_axis_name="c", subcore_axis_name="s", num_cores=NC, num_subcores=NS
)


@functools.partial(
    pl.kernel,
    mesh=_mesh,
    out_type=jax.ShapeDtypeStruct((ROWS, COLS), jnp.float32),
    scratch_types=[
        pltpu.VMEM((2, BR, BC), jnp.float32),
        pltpu.VMEM((2, BR, BC), jnp.int32),
        pltpu.VMEM((2, BR, BC), jnp.float32),
        pltpu.SemaphoreType.DMA,
        pltpu.SemaphoreType.DMA,
        pltpu.SemaphoreType.DMA,
        pltpu.SemaphoreType.DMA,
        pltpu.SemaphoreType.DMA,
        pltpu.SemaphoreType.DMA,
    ],
    compiler_params=pltpu.CompilerParams(use_tc_tiling_on_sc=True),
)
def _cordiv_sc(div_hbm, dsr_hbm, out_hbm, a_v, b_v, o_v, sa0, sa1, sb0, sb1, so0, so1):
    wid = lax.axis_index("s") * NC + lax.axis_index("c")
    row_base = wid * ROWS_W
    sa = (sa0, sa1)
    sb = (sb0, sb1)
    so = (so0, so1)
    zeros = jnp.zeros((L,), jnp.float32)

    def blk(g):
        r0 = row_base + (g // (COLS // BC)) * BR
        c0 = (g % (COLS // BC)) * BC
        return pl.ds(r0, BR), pl.ds(c0, BC)

    def in_copies(g, s):
        r, c = blk(g)
        ca = pltpu.async_copy(div_hbm.at[r, c], a_v.at[s], sa[s])
        cb = pltpu.async_copy(dsr_hbm.at[r, c], b_v.at[s], sb[s])
        return ca, cb

    def out_copy(g, s):
        r, c = blk(g)
        return pltpu.async_copy(o_v.at[s], out_hbm.at[r, c], so[s])

    pend_in = [None, None]
    pend_out = [None, None]
    pend_in[0] = in_copies(0, 0)
    for g in range(NBLK):
        s = g % 2
        if g + 1 < NBLK:
            pend_in[1 - s] = in_copies(g + 1, 1 - s)
        ca, cb = pend_in[s]
        ca.wait()
        cb.wait()
        if pend_out[s] is not None:
            pend_out[s].wait()
        a_s = a_v.at[s]
        b_s = b_v.at[s]
        o_s = o_v.at[s]

        @plsc.parallel_loop(0, BC, step=L, unroll=2)
        def _(i):
            for r in range(BR):
                d = a_s[r, pl.ds(i, L)]
                q = b_s[r, pl.ds(i, L)]
                o_s[r, pl.ds(i, L)] = jnp.where(q == 1, d, zeros)

        pend_out[s] = out_copy(g, s)

    pend_out[0].wait()
    pend_out[1].wait()


def kernel(dividend, divisor):
    return _cordiv_sc(dividend, divisor)


# speedup vs baseline: 3.6422x; 1.0093x over previous
import functools

import jax
import jax.numpy as jnp
from jax import lax
from jax.experimental import pallas as pl
from jax.experimental.pallas import tpu as pltpu
from jax.experimental.pallas import tpu_sc as plsc

NC, NS, L = 2, 16, 16
NW = NC * NS
ROWS, COLS = 2048, 4096
R_TC = 1280
R_SC = ROWS - R_TC
ROWS_W = R_SC // NW
BR, BC = 8, 2048
NBLK = (ROWS_W // BR) * (COLS // BC)
TBR = 128

_mesh = plsc.VectorSubcoreMesh(
    core_axis_name="c", subcore_axis_name="s", num_cores=NC, num_subcores=NS
)


@functools.partial(
    pl.kernel,
    mesh=_mesh,
    out_type=jax.ShapeDtypeStruct((R_SC, COLS), jnp.float32),
    scratch_types=[
        pltpu.VMEM((2, BR, BC), jnp.float32),
        pltpu.VMEM((2, BR, BC), jnp.int32),
        pltpu.VMEM((2, BR, BC), jnp.float32),
        pltpu.SemaphoreType.DMA,
        pltpu.SemaphoreType.DMA,
        pltpu.SemaphoreType.DMA,
        pltpu.SemaphoreType.DMA,
        pltpu.SemaphoreType.DMA,
        pltpu.SemaphoreType.DMA,
    ],
    compiler_params=pltpu.CompilerParams(use_tc_tiling_on_sc=True),
)
def _cordiv_sc(div_hbm, dsr_hbm, out_hbm, a_v, b_v, o_v, sa0, sa1, sb0, sb1, so0, so1):
    wid = lax.axis_index("s") * NC + lax.axis_index("c")
    row_base = wid * ROWS_W
    sa = (sa0, sa1)
    sb = (sb0, sb1)
    so = (so0, so1)
    zeros = jnp.zeros((L,), jnp.float32)

    def blk(g):
        r0 = row_base + (g // (COLS // BC)) * BR
        c0 = (g % (COLS // BC)) * BC
        return r0, pl.ds(c0, BC)

    def in_copies(g, s):
        r0, c = blk(g)
        rin = pl.ds(R_TC + r0, BR)
        ca = pltpu.async_copy(div_hbm.at[rin, c], a_v.at[s], sa[s])
        cb = pltpu.async_copy(dsr_hbm.at[rin, c], b_v.at[s], sb[s])
        return ca, cb

    def out_copy(g, s):
        r0, c = blk(g)
        return pltpu.async_copy(o_v.at[s], out_hbm.at[pl.ds(r0, BR), c], so[s])

    pend_in = [None, None]
    pend_out = [None, None]
    pend_in[0] = in_copies(0, 0)
    for g in range(NBLK):
        s = g % 2
        if g + 1 < NBLK:
            pend_in[1 - s] = in_copies(g + 1, 1 - s)
        ca, cb = pend_in[s]
        ca.wait()
        cb.wait()
        if pend_out[s] is not None:
            pend_out[s].wait()
        a_s = a_v.at[s]
        b_s = b_v.at[s]
        o_s = o_v.at[s]

        @plsc.parallel_loop(0, BC, step=L, unroll=2)
        def _(i):
            for r in range(BR):
                d = a_s[r, pl.ds(i, L)]
                q = b_s[r, pl.ds(i, L)]
                o_s[r, pl.ds(i, L)] = jnp.where(q == 1, d, zeros)

        pend_out[s] = out_copy(g, s)

    pend_out[0].wait()
    pend_out[1].wait()


def _tc_body(d_ref, q_ref, o_ref):
    o_ref[...] = jnp.where(q_ref[...] == 1, d_ref[...], 0.0)


_cordiv_tc = pl.pallas_call(
    _tc_body,
    grid=(R_TC // TBR,),
    in_specs=[
        pl.BlockSpec((TBR, COLS), lambda i: (i, 0)),
        pl.BlockSpec((TBR, COLS), lambda i: (i, 0)),
    ],
    out_specs=pl.BlockSpec((TBR, COLS), lambda i: (i, 0)),
    out_shape=jax.ShapeDtypeStruct((ROWS, COLS), jnp.float32),
)


def kernel(dividend, divisor):
    sc_out = _cordiv_sc(dividend, divisor)
    tc_out = _cordiv_tc(dividend, divisor)
    return lax.dynamic_update_slice(tc_out, sc_out, (R_TC, 0))


# speedup vs baseline: 3.9179x; 1.0757x over previous
import functools

import jax
import jax.numpy as jnp
from jax import lax
from jax.experimental import pallas as pl
from jax.experimental.pallas import tpu as pltpu
from jax.experimental.pallas import tpu_sc as plsc

NC, NS, L = 2, 16, 16
NW = NC * NS
ROWS, COLS = 2048, 4096
R_TC = 1536
R_SC = ROWS - R_TC
ROWS_W = R_SC // NW
BR, BC = 8, 2048
NBLK = (ROWS_W // BR) * (COLS // BC)
TBR = 256

_mesh = plsc.VectorSubcoreMesh(
    core_axis_name="c", subcore_axis_name="s", num_cores=NC, num_subcores=NS
)


@functools.partial(
    pl.kernel,
    mesh=_mesh,
    out_type=jax.ShapeDtypeStruct((R_SC, COLS), jnp.float32),
    scratch_types=[
        pltpu.VMEM((2, BR, BC), jnp.float32),
        pltpu.VMEM((2, BR, BC), jnp.int32),
        pltpu.VMEM((2, BR, BC), jnp.float32),
        pltpu.SemaphoreType.DMA,
        pltpu.SemaphoreType.DMA,
        pltpu.SemaphoreType.DMA,
        pltpu.SemaphoreType.DMA,
        pltpu.SemaphoreType.DMA,
        pltpu.SemaphoreType.DMA,
    ],
    compiler_params=pltpu.CompilerParams(use_tc_tiling_on_sc=True),
)
def _cordiv_sc(div_hbm, dsr_hbm, out_hbm, a_v, b_v, o_v, sa0, sa1, sb0, sb1, so0, so1):
    wid = lax.axis_index("s") * NC + lax.axis_index("c")
    row_base = wid * ROWS_W
    sa = (sa0, sa1)
    sb = (sb0, sb1)
    so = (so0, so1)
    zeros = jnp.zeros((L,), jnp.float32)

    def blk(g):
        r0 = row_base + (g // (COLS // BC)) * BR
        c0 = (g % (COLS // BC)) * BC
        return r0, pl.ds(c0, BC)

    def in_copies(g, s):
        r0, c = blk(g)
        rin = pl.ds(R_TC + r0, BR)
        ca = pltpu.async_copy(div_hbm.at[rin, c], a_v.at[s], sa[s])
        cb = pltpu.async_copy(dsr_hbm.at[rin, c], b_v.at[s], sb[s])
        return ca, cb

    def out_copy(g, s):
        r0, c = blk(g)
        return pltpu.async_copy(o_v.at[s], out_hbm.at[pl.ds(r0, BR), c], so[s])

    pend_in = [None, None]
    pend_out = [None, None]
    pend_in[0] = in_copies(0, 0)
    for g in range(NBLK):
        s = g % 2
        if g + 1 < NBLK:
            pend_in[1 - s] = in_copies(g + 1, 1 - s)
        ca, cb = pend_in[s]
        ca.wait()
        cb.wait()
        if pend_out[s] is not None:
            pend_out[s].wait()
        a_s = a_v.at[s]
        b_s = b_v.at[s]
        o_s = o_v.at[s]

        @plsc.parallel_loop(0, BC, step=L, unroll=2)
        def _(i):
            for r in range(BR):
                d = a_s[r, pl.ds(i, L)]
                q = b_s[r, pl.ds(i, L)]
                o_s[r, pl.ds(i, L)] = jnp.where(q == 1, d, zeros)

        pend_out[s] = out_copy(g, s)

    pend_out[0].wait()
    pend_out[1].wait()


def _tc_body(d_ref, q_ref, o_ref):
    o_ref[...] = jnp.where(q_ref[...] == 1, d_ref[...], 0.0)


_cordiv_tc = pl.pallas_call(
    _tc_body,
    grid=(R_TC // TBR,),
    in_specs=[
        pl.BlockSpec((TBR, COLS), lambda i: (i, 0)),
        pl.BlockSpec((TBR, COLS), lambda i: (i, 0)),
    ],
    out_specs=pl.BlockSpec((TBR, COLS), lambda i: (i, 0)),
    out_shape=jax.ShapeDtypeStruct((ROWS, COLS), jnp.float32),
)


def kernel(dividend, divisor):
    sc_out = _cordiv_sc(dividend, divisor)
    tc_out = _cordiv_tc(dividend, divisor)
    return lax.dynamic_update_slice(tc_out, sc_out, (R_TC, 0))


# speedup vs baseline: 4.1888x; 1.0691x over previous
import functools

import jax
import jax.numpy as jnp
from jax import lax
from jax.experimental import pallas as pl
from jax.experimental.pallas import tpu as pltpu
from jax.experimental.pallas import tpu_sc as plsc

NC, NS, L = 2, 16, 16
NW = NC * NS
ROWS, COLS = 2048, 4096
R_TC = 1792
R_SC = ROWS - R_TC
ROWS_W = R_SC // NW
BR, BC = 8, 1024
NBLK = (ROWS_W // BR) * (COLS // BC)
NSLOT = 4
TBR = 256

_mesh = plsc.VectorSubcoreMesh(
    core_axis_name="c", subcore_axis_name="s", num_cores=NC, num_subcores=NS
)


@functools.partial(
    pl.kernel,
    mesh=_mesh,
    out_type=jax.ShapeDtypeStruct((R_SC, COLS), jnp.float32),
    scratch_types=[
        pltpu.VMEM((NSLOT, BR, BC), jnp.float32),
        pltpu.VMEM((NSLOT, BR, BC), jnp.int32),
        pltpu.VMEM((NSLOT, BR, BC), jnp.float32),
    ]
    + [pltpu.SemaphoreType.DMA] * (3 * NSLOT),
    compiler_params=pltpu.CompilerParams(use_tc_tiling_on_sc=True),
)
def _cordiv_sc(div_hbm, dsr_hbm, out_hbm, a_v, b_v, o_v, *sems):
    wid = lax.axis_index("s") * NC + lax.axis_index("c")
    row_base = wid * ROWS_W
    sa = sems[0:NSLOT]
    sb = sems[NSLOT : 2 * NSLOT]
    so = sems[2 * NSLOT : 3 * NSLOT]
    zeros = jnp.zeros((L,), jnp.float32)

    def blk(g):
        r0 = row_base + (g // (COLS // BC)) * BR
        c0 = (g % (COLS // BC)) * BC
        return r0, pl.ds(c0, BC)

    def in_copies(g, s):
        r0, c = blk(g)
        rin = pl.ds(R_TC + r0, BR)
        ca = pltpu.async_copy(div_hbm.at[rin, c], a_v.at[s], sa[s])
        cb = pltpu.async_copy(dsr_hbm.at[rin, c], b_v.at[s], sb[s])
        return ca, cb

    def out_copy(g, s):
        r0, c = blk(g)
        return pltpu.async_copy(o_v.at[s], out_hbm.at[pl.ds(r0, BR), c], so[s])

    pend_in = [None] * NSLOT
    pend_out = [None] * NSLOT
    for g in range(min(NSLOT, NBLK)):
        pend_in[g] = in_copies(g, g)
    for g in range(NBLK):
        s = g % NSLOT
        ca, cb = pend_in[s]
        ca.wait()
        cb.wait()
        if pend_out[s] is not None:
            pend_out[s].wait()
        a_s = a_v.at[s]
        b_s = b_v.at[s]
        o_s = o_v.at[s]

        @plsc.parallel_loop(0, BC, step=L, unroll=2)
        def _(i):
            for r in range(BR):
                d = a_s[r, pl.ds(i, L)]
                q = b_s[r, pl.ds(i, L)]
                o_s[r, pl.ds(i, L)] = jnp.where(q == 1, d, zeros)

        pend_out[s] = out_copy(g, s)
        if g + NSLOT < NBLK:
            pend_in[s] = in_copies(g + NSLOT, s)

    for s in range(NSLOT):
        if pend_out[s] is not None:
            pend_out[s].wait()


def _tc_body(d_ref, q_ref, o_ref):
    o_ref[...] = jnp.where(q_ref[...] == 1, d_ref[...], 0.0)


_cordiv_tc = pl.pallas_call(
    _tc_body,
    grid=(R_TC // TBR,),
    in_specs=[
        pl.BlockSpec((TBR, COLS), lambda i: (i, 0)),
        pl.BlockSpec((TBR, COLS), lambda i: (i, 0)),
    ],
    out_specs=pl.BlockSpec((TBR, COLS), lambda i: (i, 0)),
    out_shape=jax.ShapeDtypeStruct((ROWS, COLS), jnp.float32),
)


def kernel(dividend, divisor):
    sc_out = _cordiv_sc(dividend, divisor)
    tc_out = _cordiv_tc(dividend, divisor)
    return lax.dynamic_update_slice(tc_out, sc_out, (R_TC, 0))
